# baseline (speedup 1.0000x reference)
import functools

import jax
import jax.numpy as jnp
from jax import lax
from jax.experimental import pallas as pl
from jax.experimental.pallas import tpu as pltpu
from jax.experimental.pallas import tpu_sc as plsc

B = 16384
D = 32

_INFO = plsc.get_sparse_core_info()
NC = _INFO.num_cores
NS = _INFO.num_subcores
L = _INFO.num_lanes
NW = NC * NS
CHUNK = B // NW
JCH = 128
NJ = CHUNK // JCH
GA = 8

LN2 = 0.6931471805599453
SQRT2 = 1.4142135623730951


@functools.partial(
    pl.kernel,
    out_type=jax.ShapeDtypeStruct((B,), jnp.float32),
    mesh=plsc.VectorSubcoreMesh(core_axis_name="c", subcore_axis_name="s"),
    compiler_params=pltpu.CompilerParams(
        needs_layout_passes=False, use_tc_tiling_on_sc=False),
    scratch_types=[
        pltpu.VMEM((CHUNK,), jnp.int32),
        pltpu.VMEM((CHUNK,), jnp.int32),
        pltpu.VMEM((CHUNK,), jnp.float32),
        pltpu.VMEM((D * CHUNK,), jnp.float32),
        pltpu.VMEM((D * CHUNK,), jnp.float32),
        pltpu.VMEM((CHUNK,), jnp.float32),
        pltpu.VMEM((CHUNK,), jnp.float32),
        pltpu.VMEM((CHUNK,), jnp.float32),
        pltpu.VMEM((L,), jnp.float32),
        pltpu.VMEM((L,), jnp.float32),
        pltpu.SemaphoreType.DMA,
        pltpu.SemaphoreType.DMA,
        pltpu.SemaphoreType.DMA,
    ],
)
def _mfpoly2_sc(u_hbm, i_hbm, f_hbm, ub_hbm, uvT_hbm, ib_hbm, ivT_hbm,
                c0_hbm, c1_hbm, out_hbm,
                idx_u, idx_i, f_v, vu, vi, bu, bi, o_v, c0_v, c1_v,
                sem_b, sem_u, sem_i):
    wid = lax.axis_index("s") * NC + lax.axis_index("c")
    base = pl.multiple_of(wid * CHUNK, CHUNK)

    pltpu.sync_copy(u_hbm.at[pl.ds(base, CHUNK)], idx_u)
    pltpu.sync_copy(i_hbm.at[pl.ds(base, CHUNK)], idx_i)
    pltpu.sync_copy(f_hbm.at[pl.ds(base, CHUNK)], f_v)
    pltpu.sync_copy(c0_hbm, c0_v)
    pltpu.sync_copy(c1_hbm, c1_v)

    bias_copies = []
    for j in range(NJ):
        sl = pl.ds(j * JCH, JCH)
        bias_copies.append(
            pltpu.async_copy(ub_hbm.at[idx_u.at[sl]], bu.at[sl], sem_b))
        bias_copies.append(
            pltpu.async_copy(ib_hbm.at[idx_i.at[sl]], bi.at[sl], sem_b))

    def fwait(tab_hbm, dstbuf, sem):
        pltpu.make_async_copy(
            tab_hbm.at[0].at[idx_u.at[pl.ds(0, JCH)]],
            dstbuf.at[pl.ds(0, JCH)], sem).wait()

    nfired = 0
    for c in range(D):
        for j in range(NJ):
            sl = pl.ds(j * JCH, JCH)
            dsl = pl.ds(c * CHUNK + j * JCH, JCH)
            if nfired >= GA:
                fwait(uvT_hbm, vu, sem_u)
                fwait(ivT_hbm, vi, sem_i)
            pltpu.make_async_copy(
                uvT_hbm.at[c].at[idx_u.at[sl]], vu.at[dsl], sem_u).start()
            pltpu.make_async_copy(
                ivT_hbm.at[c].at[idx_i.at[sl]], vi.at[dsl], sem_i).start()
            nfired += 1
    for _ in range(GA):
        fwait(uvT_hbm, vu, sem_u)
        fwait(ivT_hbm, vi, sem_i)
    for c_ in bias_copies:
        c_.wait()

    c0s = c0_v[...]
    c1s = c1_v[...]

    def group(g, _):
        gb = pl.multiple_of(g * L, L)
        acc = jnp.zeros((L,), jnp.float32)
        for d in range(D):
            xu = vu[pl.ds(d * CHUNK + gb, L)]
            xi = vi[pl.ds(d * CHUNK + gb, L)]
            acc = acc + xu * xi

        sl = pl.ds(gb, L)
        fg = f_v[sl]
        xb = plsc.bitcast(fg, jnp.int32)
        e = lax.shift_right_logical(xb, 23) - 127
        m = plsc.bitcast((xb & 0x7FFFFF) | (127 << 23), jnp.float32)
        big = m > SQRT2
        m = jnp.where(big, m * 0.5, m)
        e = jnp.where(big, e + 1, e)
        s = (m - 1.0) / (m + 1.0)
        z = s * s
        ln_m = s * (2.0 + z * (2.0 / 3.0 + z * (2.0 / 5.0
                    + z * (2.0 / 7.0 + z * (2.0 / 9.0)))))
        logf = ln_m + e.astype(jnp.float32) * LN2

        o_v[sl] = acc + bu[sl] + bi[sl] + c1s * logf + c0s
        return 0

    lax.fori_loop(0, CHUNK // L, group, 0)

    pltpu.sync_copy(o_v, out_hbm.at[pl.ds(base, CHUNK)])


def kernel(u, i, f, glob_bias, user_bias, user_vec, item_bias, item_vec,
           poly_W, poly_b):
    u = jnp.squeeze(u).astype(jnp.int32)
    i = jnp.squeeze(i).astype(jnp.int32)
    f = jnp.squeeze(f).astype(jnp.float32)
    uvT = jnp.transpose(user_vec)
    ivT = jnp.transpose(item_vec)
    c1 = jnp.full((L,), poly_W[0, 0] + 2.0 * poly_W[0, 1], jnp.float32)
    c0 = jnp.full((L,), poly_b[0] + glob_bias[0], jnp.float32)
    return _mfpoly2_sc(u, i, f, user_bias, uvT, item_bias, ivT, c0, c1)


# speedup vs baseline: 5.5318x; 5.5318x over previous
import functools

import jax
import jax.numpy as jnp
from jax import lax
from jax.experimental import pallas as pl
from jax.experimental.pallas import tpu as pltpu
from jax.experimental.pallas import tpu_sc as plsc

B = 16384
D = 32

_INFO = plsc.get_sparse_core_info()
NC = _INFO.num_cores
NS = _INFO.num_subcores
NW = NC * NS
CHUNK = B // NW
JCH = 128
NJ = CHUNK // JCH


@functools.partial(
    pl.kernel,
    out_type=(
        jax.ShapeDtypeStruct((B, D), jnp.float32),
        jax.ShapeDtypeStruct((B, D), jnp.float32),
        jax.ShapeDtypeStruct((B,), jnp.float32),
        jax.ShapeDtypeStruct((B,), jnp.float32),
    ),
    mesh=plsc.VectorSubcoreMesh(core_axis_name="c", subcore_axis_name="s"),
    compiler_params=pltpu.CompilerParams(
        needs_layout_passes=False, use_tc_tiling_on_sc=False),
    scratch_types=[
        pltpu.VMEM((CHUNK,), jnp.int32),
        pltpu.VMEM((CHUNK,), jnp.int32),
        pltpu.VMEM((CHUNK, D), jnp.float32),
        pltpu.VMEM((CHUNK, D), jnp.float32),
        pltpu.VMEM((CHUNK,), jnp.float32),
        pltpu.VMEM((CHUNK,), jnp.float32),
        pltpu.SemaphoreType.DMA,
        pltpu.SemaphoreType.DMA,
        pltpu.SemaphoreType.DMA,
    ],
)
def _gather_sc(u_hbm, i_hbm, ub_hbm, uv_hbm, ib_hbm, iv_hbm,
               vu_out, vi_out, bu_out, bi_out,
               idx_u, idx_i, vu, vi, bu, bi, sem_u, sem_i, sem_b):
    wid = lax.axis_index("s") * NC + lax.axis_index("c")
    base = pl.multiple_of(wid * CHUNK, CHUNK)

    pltpu.sync_copy(u_hbm.at[pl.ds(base, CHUNK)], idx_u)
    pltpu.sync_copy(i_hbm.at[pl.ds(base, CHUNK)], idx_i)

    copies = []
    for j in range(NJ):
        sl = pl.ds(j * JCH, JCH)
        copies.append(
            pltpu.async_copy(uv_hbm.at[idx_u.at[sl]], vu.at[sl], sem_u))
        copies.append(
            pltpu.async_copy(iv_hbm.at[idx_i.at[sl]], vi.at[sl], sem_i))
        copies.append(
            pltpu.async_copy(ub_hbm.at[idx_u.at[sl]], bu.at[sl], sem_b))
        copies.append(
            pltpu.async_copy(ib_hbm.at[idx_i.at[sl]], bi.at[sl], sem_b))
    for c in copies:
        c.wait()

    pltpu.sync_copy(vu, vu_out.at[pl.ds(base, CHUNK)])
    pltpu.sync_copy(vi, vi_out.at[pl.ds(base, CHUNK)])
    pltpu.sync_copy(bu, bu_out.at[pl.ds(base, CHUNK)])
    pltpu.sync_copy(bi, bi_out.at[pl.ds(base, CHUNK)])


def _dense_tc(c_ref, vu_ref, vi_ref, bu_ref, bi_ref, f_ref, o_ref):
    c0 = c_ref[0]
    c1 = c_ref[1]
    intx = jnp.sum(vu_ref[...] * vi_ref[...], axis=1)
    o_ref[...] = (intx + bu_ref[...] + bi_ref[...]
                  + c1 * jnp.log(f_ref[...]) + c0)


def kernel(u, i, f, glob_bias, user_bias, user_vec, item_bias, item_vec,
           poly_W, poly_b):
    u = jnp.squeeze(u).astype(jnp.int32)
    i = jnp.squeeze(i).astype(jnp.int32)
    f = jnp.squeeze(f).astype(jnp.float32)

    vu_g, vi_g, bu_g, bi_g = _gather_sc(
        u, i, user_bias, user_vec, item_bias, item_vec)

    c = jnp.stack([poly_b[0] + glob_bias[0],
                   poly_W[0, 0] + 2.0 * poly_W[0, 1]])

    return pl.pallas_call(
        _dense_tc,
        out_shape=jax.ShapeDtypeStruct((B,), jnp.float32),
        in_specs=[pl.BlockSpec(memory_space=pltpu.SMEM)]
        + [pl.BlockSpec(memory_space=pltpu.VMEM)] * 5,
        out_specs=pl.BlockSpec(memory_space=pltpu.VMEM),
    )(c, vu_g, vi_g, bu_g, bi_g, f)
